# baseline (speedup 1.0000x reference)
import jax
import jax.numpy as jnp
from jax.experimental import pallas as pl
from jax.experimental.pallas import tpu as pltpu
from jax.experimental.xla_metadata import set_xla_metadata

_N = 256
_M = 51529
_CK = 3968
_NC = 13
_MPAD = _CK * _NC
_BB = 2
_QC = 16


def _glcm_body(a_ref, b_ref, pa_ref, pb_ref, out_ref):
    reps = _CK // 128
    pa = jnp.concatenate([pa_ref[...]] * reps, axis=1)
    pb = jnp.concatenate([pb_ref[...]] * reps, axis=1)
    for bb in range(_BB):
        acc = jnp.zeros((_N, _N), jnp.float32)
        for c in range(_NC):
            a_row = a_ref[bb, :, c * _CK:(c + 1) * _CK]
            b_row = b_ref[bb, :, c * _CK:(c + 1) * _CK]
            sa = (jnp.broadcast_to(a_row, (_N, _CK)) - pa).astype(jnp.bfloat16)
            sb = (jnp.broadcast_to(b_row, (_N, _CK)) - pb).astype(jnp.bfloat16)
            sa = jnp.clip(sa, 0.0, 1.0)
            sb = jnp.clip(sb, 0.0, 1.0)
            acc = acc + jax.lax.dot_general(
                sb, sa, (((1,), (1,)), ((), ())),
                preferred_element_type=jnp.float32)
        out_ref[bb] = acc


def _linear_body(g_ref, w_ref, bias_ref, out_ref, acc_ref):
    c = pl.program_id(0)
    p = jnp.zeros((32, 128), jnp.float32)
    for ql in range(_QC):
        lhs = g_ref[:, 4 * ql:4 * ql + 4, :].reshape(32, _N)
        wq = w_ref[_N * ql:_N * (ql + 1), :].astype(jnp.float32)
        p = p + jax.lax.dot_general(lhs, wq, (((1,), (0,)), ((), ())),
                                    preferred_element_type=jnp.float32)

    @pl.when(c == 0)
    def _():
        acc_ref[...] = p

    @pl.when(c > 0)
    def _():
        acc_ref[...] = acc_ref[...] + p

    @pl.when(c == 64 // _QC - 1)
    def _():
        r8 = acc_ref[...].reshape(8, 4, 128)
        s = (r8[:, 0, 0:32] + r8[:, 1, 32:64]
             + r8[:, 2, 64:96] + r8[:, 3, 96:128])
        out_ref[...] = jnp.maximum(s + bias_ref[...], 0.0)


def kernel(x, phi_a, phi_b, weight, bias):
    b_sz = x.shape[0]
    a = x.reshape(b_sz, -1)
    bdiff = a - jnp.pad(a[:, 1:], ((0, 0), (0, 1)))
    pad = _MPAD - _M
    a_p = jnp.pad(a, ((0, 0), (0, pad))).reshape(b_sz, 1, _MPAD)
    b_p = jnp.pad(bdiff, ((0, 0), (0, pad))).reshape(b_sz, 1, _MPAD)
    pa128 = jnp.broadcast_to(phi_a[:, None], (_N, 128))
    pb128 = jnp.broadcast_to(phi_b[:, None], (_N, 128))

    glcm_call = pl.pallas_call(
        _glcm_body,
        grid=(b_sz // _BB,),
        in_specs=[
            pl.BlockSpec((_BB, 1, _MPAD), lambda b: (b, 0, 0)),
            pl.BlockSpec((_BB, 1, _MPAD), lambda b: (b, 0, 0)),
            pl.BlockSpec((_N, 128), lambda b: (0, 0)),
            pl.BlockSpec((_N, 128), lambda b: (0, 0)),
        ],
        out_specs=pl.BlockSpec((_BB, _N, _N), lambda b: (b, 0, 0)),
        out_shape=jax.ShapeDtypeStruct((b_sz, _N, _N), jnp.float32),
        compiler_params=pltpu.CompilerParams(
            dimension_semantics=(pltpu.PARALLEL,),
        ),
    )
    with set_xla_metadata(_scheduling_group_id=0):
        glcm_t = glcm_call(a_p, b_p, pa128, pb128)
        w_perm = weight.astype(jnp.bfloat16).reshape(
            _N, 64, 4, 32).transpose(1, 0, 2, 3).reshape(64 * _N, 128)

    out = pl.pallas_call(
        _linear_body,
        grid=(64 // _QC,),
        in_specs=[
            pl.BlockSpec((b_sz, 4 * _QC, _N), lambda c: (0, c, 0)),
            pl.BlockSpec((_N * _QC, 128), lambda c: (c, 0)),
            pl.BlockSpec((1, 32), lambda c: (0, 0)),
        ],
        out_specs=pl.BlockSpec((b_sz, 32), lambda c: (0, 0)),
        out_shape=jax.ShapeDtypeStruct((b_sz, 32), jnp.float32),
        scratch_shapes=[pltpu.VMEM((32, 128), jnp.float32)],
        compiler_params=pltpu.CompilerParams(
            dimension_semantics=(pltpu.ARBITRARY,),
        ),
    )(glcm_t, w_perm, bias.reshape(1, 32))
    return out


# speedup vs baseline: 1.0022x; 1.0022x over previous
import jax
import jax.numpy as jnp
from jax.experimental import pallas as pl
from jax.experimental.pallas import tpu as pltpu
from jax.experimental.xla_metadata import set_xla_metadata

_N = 256
_M = 51529
_CK = 3968
_NC = 13
_MPAD = _CK * _NC
_BB = 2
_QC = 16


def _glcm_body(a_ref, b_ref, pa_ref, pb_ref, out_ref):
    reps = _CK // 128
    pa = jnp.concatenate([pa_ref[...]] * reps, axis=1)
    pb = jnp.concatenate([pb_ref[...]] * reps, axis=1)
    for bb in range(_BB):
        acc = jnp.zeros((_N, _N), jnp.float32)
        for c in range(_NC):
            a_row = a_ref[bb, :, c * _CK:(c + 1) * _CK]
            b_row = b_ref[bb, :, c * _CK:(c + 1) * _CK]
            sa = (jnp.broadcast_to(a_row, (_N, _CK)) - pa).astype(jnp.bfloat16)
            sb = (jnp.broadcast_to(b_row, (_N, _CK)) - pb).astype(jnp.bfloat16)
            sa = jnp.clip(sa, 0.0, 1.0)
            sb = jnp.clip(sb, 0.0, 1.0)
            acc = acc + jax.lax.dot_general(
                sb, sa, (((1,), (1,)), ((), ())),
                preferred_element_type=jnp.float32)
        out_ref[bb] = acc


def _linear_body(g_ref, w_ref, bias_ref, out_ref, acc_ref):
    c = pl.program_id(0)
    p = jnp.zeros((32, 128), jnp.float32)
    for ql in range(_QC):
        lhs = g_ref[:, 4 * ql:4 * ql + 4, :].reshape(32, _N)
        wq = w_ref[_N * ql:_N * (ql + 1), :]
        p = p + jax.lax.dot_general(lhs, wq, (((1,), (0,)), ((), ())),
                                    preferred_element_type=jnp.float32)

    @pl.when(c == 0)
    def _():
        acc_ref[...] = p

    @pl.when(c > 0)
    def _():
        acc_ref[...] = acc_ref[...] + p

    @pl.when(c == 64 // _QC - 1)
    def _():
        r8 = acc_ref[...].reshape(8, 4, 128)
        s = (r8[:, 0, 0:32] + r8[:, 1, 32:64]
             + r8[:, 2, 64:96] + r8[:, 3, 96:128])
        out_ref[...] = jnp.maximum(s + bias_ref[...], 0.0)


def kernel(x, phi_a, phi_b, weight, bias):
    b_sz = x.shape[0]
    a = x.reshape(b_sz, -1)
    bdiff = a - jnp.pad(a[:, 1:], ((0, 0), (0, 1)))
    pad = _MPAD - _M
    a_p = jnp.pad(a, ((0, 0), (0, pad))).reshape(b_sz, 1, _MPAD)
    b_p = jnp.pad(bdiff, ((0, 0), (0, pad))).reshape(b_sz, 1, _MPAD)
    pa128 = jnp.broadcast_to(phi_a[:, None], (_N, 128))
    pb128 = jnp.broadcast_to(phi_b[:, None], (_N, 128))

    glcm_call = pl.pallas_call(
        _glcm_body,
        grid=(b_sz // _BB,),
        in_specs=[
            pl.BlockSpec((_BB, 1, _MPAD), lambda b: (b, 0, 0)),
            pl.BlockSpec((_BB, 1, _MPAD), lambda b: (b, 0, 0)),
            pl.BlockSpec((_N, 128), lambda b: (0, 0)),
            pl.BlockSpec((_N, 128), lambda b: (0, 0)),
        ],
        out_specs=pl.BlockSpec((_BB, _N, _N), lambda b: (b, 0, 0)),
        out_shape=jax.ShapeDtypeStruct((b_sz, _N, _N), jnp.float32),
        compiler_params=pltpu.CompilerParams(
            dimension_semantics=(pltpu.PARALLEL,),
        ),
    )
    with set_xla_metadata(_scheduling_group_id=0):
        glcm_t = glcm_call(a_p, b_p, pa128, pb128)
        w_perm = weight.reshape(_N, 64, 4, 32).transpose(1, 0, 2, 3).reshape(
            64 * _N, 128)

    out = pl.pallas_call(
        _linear_body,
        grid=(64 // _QC,),
        in_specs=[
            pl.BlockSpec((b_sz, 4 * _QC, _N), lambda c: (0, c, 0)),
            pl.BlockSpec((_N * _QC, 128), lambda c: (c, 0)),
            pl.BlockSpec((1, 32), lambda c: (0, 0)),
        ],
        out_specs=pl.BlockSpec((b_sz, 32), lambda c: (0, 0)),
        out_shape=jax.ShapeDtypeStruct((b_sz, 32), jnp.float32),
        scratch_shapes=[pltpu.VMEM((32, 128), jnp.float32)],
        compiler_params=pltpu.CompilerParams(
            dimension_semantics=(pltpu.ARBITRARY,),
        ),
    )(glcm_t, w_perm, bias.reshape(1, 32))
    return out


# speedup vs baseline: 1.0399x; 1.0376x over previous
import jax
import jax.numpy as jnp
from jax.experimental import pallas as pl
from jax.experimental.pallas import tpu as pltpu

_N = 256
_M = 51529
_CK = 3968
_NC = 13
_MPAD = _CK * _NC
_BB = 2
_QC = 16


def _glcm_body(a_ref, b_ref, pa_ref, pb_ref, out_ref):
    reps = _CK // 128
    pa = jnp.concatenate([pa_ref[...]] * reps, axis=1)
    pb = jnp.concatenate([pb_ref[...]] * reps, axis=1)
    for bb in range(_BB):
        acc = jnp.zeros((_N, _N), jnp.float32)
        for c in range(_NC):
            a_row = a_ref[bb, :, c * _CK:(c + 1) * _CK]
            b_row = b_ref[bb, :, c * _CK:(c + 1) * _CK]
            sa = (jnp.broadcast_to(a_row, (_N, _CK)) - pa).astype(jnp.bfloat16)
            sb = (jnp.broadcast_to(b_row, (_N, _CK)) - pb).astype(jnp.bfloat16)
            sa = jnp.clip(sa, 0.0, 1.0)
            sb = jnp.clip(sb, 0.0, 1.0)
            acc = acc + jax.lax.dot_general(
                sb, sa, (((1,), (1,)), ((), ())),
                preferred_element_type=jnp.float32)
        out_ref[bb] = acc


def _linear_body(g_ref, w_ref, bias_ref, out_ref, acc_ref):
    c = pl.program_id(0)
    p = jnp.zeros((32, 128), jnp.float32)
    for ql in range(_QC):
        lhs = g_ref[:, 4 * ql:4 * ql + 4, :].reshape(32, _N)
        wq = w_ref[:, ql, :]
        p = p + jax.lax.dot_general(lhs, wq, (((1,), (0,)), ((), ())),
                                    preferred_element_type=jnp.float32)

    @pl.when(c == 0)
    def _():
        acc_ref[...] = p

    @pl.when(c > 0)
    def _():
        acc_ref[...] = acc_ref[...] + p

    @pl.when(c == 64 // _QC - 1)
    def _():
        r8 = acc_ref[...].reshape(8, 4, 128)
        s = (r8[:, 0, 0:32] + r8[:, 1, 32:64]
             + r8[:, 2, 64:96] + r8[:, 3, 96:128])
        out_ref[...] = jnp.maximum(s + bias_ref[...], 0.0)


def kernel(x, phi_a, phi_b, weight, bias):
    b_sz = x.shape[0]
    a = x.reshape(b_sz, -1)
    bdiff = a - jnp.pad(a[:, 1:], ((0, 0), (0, 1)))
    pad = _MPAD - _M
    a_p = jnp.pad(a, ((0, 0), (0, pad))).reshape(b_sz, 1, _MPAD)
    b_p = jnp.pad(bdiff, ((0, 0), (0, pad))).reshape(b_sz, 1, _MPAD)
    pa128 = jnp.broadcast_to(phi_a[:, None], (_N, 128))
    pb128 = jnp.broadcast_to(phi_b[:, None], (_N, 128))

    glcm_call = pl.pallas_call(
        _glcm_body,
        grid=(b_sz // _BB,),
        in_specs=[
            pl.BlockSpec((_BB, 1, _MPAD), lambda b: (b, 0, 0)),
            pl.BlockSpec((_BB, 1, _MPAD), lambda b: (b, 0, 0)),
            pl.BlockSpec((_N, 128), lambda b: (0, 0)),
            pl.BlockSpec((_N, 128), lambda b: (0, 0)),
        ],
        out_specs=pl.BlockSpec((_BB, _N, _N), lambda b: (b, 0, 0)),
        out_shape=jax.ShapeDtypeStruct((b_sz, _N, _N), jnp.float32),
        compiler_params=pltpu.CompilerParams(
            dimension_semantics=(pltpu.PARALLEL,),
        ),
    )
    glcm_t = glcm_call(a_p, b_p, pa128, pb128)
    w3 = weight.reshape(_N, 64, 128)

    out = pl.pallas_call(
        _linear_body,
        grid=(64 // _QC,),
        in_specs=[
            pl.BlockSpec((b_sz, 4 * _QC, _N), lambda c: (0, c, 0)),
            pl.BlockSpec((_N, _QC, 128), lambda c: (0, c, 0)),
            pl.BlockSpec((1, 32), lambda c: (0, 0)),
        ],
        out_specs=pl.BlockSpec((b_sz, 32), lambda c: (0, 0)),
        out_shape=jax.ShapeDtypeStruct((b_sz, 32), jnp.float32),
        scratch_shapes=[pltpu.VMEM((32, 128), jnp.float32)],
        compiler_params=pltpu.CompilerParams(
            dimension_semantics=(pltpu.ARBITRARY,),
        ),
    )(glcm_t, w3, bias.reshape(1, 32))
    return out


# speedup vs baseline: 1.1259x; 1.0827x over previous
import jax
import jax.numpy as jnp
from jax.experimental import pallas as pl
from jax.experimental.pallas import tpu as pltpu

_N = 256
_M = 51529
_CK = 3968
_NC = 13
_MPAD = _CK * _NC
_BB = 2
_KOUT = 65536
_W_CHUNK = 16384


def _glcm_body(a_ref, b_ref, pa_ref, pb_ref, out_ref):
    reps = _CK // 128
    pa = jnp.concatenate([pa_ref[...]] * reps, axis=1)
    pb = jnp.concatenate([pb_ref[...]] * reps, axis=1)
    for bb in range(_BB):
        acc = jnp.zeros((_N, _N), jnp.float32)
        for c in range(_NC):
            a_row = a_ref[bb, :, c * _CK:(c + 1) * _CK]
            b_row = b_ref[bb, :, c * _CK:(c + 1) * _CK]
            sa = (jnp.broadcast_to(a_row, (_N, _CK)) - pa).astype(jnp.bfloat16)
            sb = (jnp.broadcast_to(b_row, (_N, _CK)) - pb).astype(jnp.bfloat16)
            sa = jnp.clip(sa, 0.0, 1.0)
            sb = jnp.clip(sb, 0.0, 1.0)
            acc = acc + jax.lax.dot_general(
                sa, sb, (((1,), (1,)), ((), ())),
                preferred_element_type=jnp.float32)
        out_ref[bb] = acc


def _linear_body(g_ref, w_ref, bias_ref, out_ref):
    c = pl.program_id(0)
    g = g_ref[...]
    w = w_ref[...]
    h = _W_CHUNK // 2
    p = jax.lax.dot_general(g[:, :h], w[:h, :], (((1,), (0,)), ((), ())),
                            preferred_element_type=jnp.float32)
    p = p + jax.lax.dot_general(g[:, h:], w[h:, :], (((1,), (0,)), ((), ())),
                                preferred_element_type=jnp.float32)

    @pl.when(c == 0)
    def _():
        out_ref[...] = p

    @pl.when(c > 0)
    def _():
        out_ref[...] = out_ref[...] + p

    @pl.when(c == (_KOUT // _W_CHUNK) - 1)
    def _():
        out_ref[...] = jnp.maximum(out_ref[...] + bias_ref[...], 0.0)


def kernel(x, phi_a, phi_b, weight, bias):
    b_sz = x.shape[0]
    a = x.reshape(b_sz, -1)
    bdiff = a - jnp.pad(a[:, 1:], ((0, 0), (0, 1)))
    pad = _MPAD - _M
    a_p = jnp.pad(a, ((0, 0), (0, pad))).reshape(b_sz, 1, _MPAD)
    b_p = jnp.pad(bdiff, ((0, 0), (0, pad))).reshape(b_sz, 1, _MPAD)
    pa128 = jnp.broadcast_to(phi_a[:, None], (_N, 128))
    pb128 = jnp.broadcast_to(phi_b[:, None], (_N, 128))

    glcm = pl.pallas_call(
        _glcm_body,
        grid=(b_sz // _BB,),
        in_specs=[
            pl.BlockSpec((_BB, 1, _MPAD), lambda b: (b, 0, 0)),
            pl.BlockSpec((_BB, 1, _MPAD), lambda b: (b, 0, 0)),
            pl.BlockSpec((_N, 128), lambda b: (0, 0)),
            pl.BlockSpec((_N, 128), lambda b: (0, 0)),
        ],
        out_specs=pl.BlockSpec((_BB, _N, _N), lambda b: (b, 0, 0)),
        out_shape=jax.ShapeDtypeStruct((b_sz, _N, _N), jnp.float32),
        compiler_params=pltpu.CompilerParams(
            dimension_semantics=(pltpu.PARALLEL,),
        ),
    )(a_p, b_p, pa128, pb128)

    g = glcm.reshape(b_sz, _KOUT)
    nsteps = _KOUT // _W_CHUNK
    out = pl.pallas_call(
        _linear_body,
        grid=(nsteps,),
        in_specs=[
            pl.BlockSpec((b_sz, _W_CHUNK), lambda c: (0, c)),
            pl.BlockSpec((_W_CHUNK, 32), lambda c: (c, 0)),
            pl.BlockSpec((1, 32), lambda c: (0, 0)),
        ],
        out_specs=pl.BlockSpec((b_sz, 32), lambda c: (0, 0)),
        out_shape=jax.ShapeDtypeStruct((b_sz, 32), jnp.float32),
        compiler_params=pltpu.CompilerParams(
            dimension_semantics=(pltpu.ARBITRARY,),
        ),
    )(g, weight, bias.reshape(1, 32))
    return out
